# baseline (speedup 1.0000x reference)
import functools

import jax
import jax.numpy as jnp
from jax import lax
from jax.experimental import pallas as pl
from jax.experimental.pallas import tpu as pltpu
from jax.experimental.pallas import tpu_sc as plsc

VOCAB = 100000
EMBED_D = 16
BATCH = 1024

V_BLK = 2048


@functools.cache
def _build_sc_gather():
    info = plsc.get_sparse_core_info()
    nc, ns = info.num_cores, info.num_subcores
    nw = nc * ns
    b_per_w = BATCH // nw

    mesh = plsc.VectorSubcoreMesh(core_axis_name="c", subcore_axis_name="s")

    @functools.partial(
        pl.kernel,
        mesh=mesh,
        out_type=jax.ShapeDtypeStruct((BATCH, EMBED_D), jnp.float32),
        scratch_types=[
            pltpu.VMEM((b_per_w,), jnp.int32),
            pltpu.VMEM((b_per_w, EMBED_D), jnp.float32),
            pltpu.SemaphoreType.DMA,
        ],
        compiler_params=pltpu.CompilerParams(use_tc_tiling_on_sc=False),
    )
    def gather(table_hbm, idx_hbm, out_hbm, idx_v, rows_v, sem):
        wid = lax.axis_index("s") * nc + lax.axis_index("c")
        base = wid * b_per_w
        pltpu.sync_copy(idx_hbm.at[pl.ds(base, b_per_w)], idx_v)
        pltpu.async_copy(table_hbm.at[idx_v], rows_v, sem).wait()
        pltpu.sync_copy(rows_v, out_hbm.at[pl.ds(base, b_per_w)])

    return gather


def _matmul_body(emb_ref, vec_ref, out_ref):
    out_ref[...] = lax.dot_general(
        vec_ref[...],
        emb_ref[...],
        (((1,), (1,)), ((), ())),
        preferred_element_type=jnp.float32,
    )


def _projection(embeddings, embed):
    grid = (pl.cdiv(VOCAB, V_BLK),)
    return pl.pallas_call(
        _matmul_body,
        grid=grid,
        in_specs=[
            pl.BlockSpec((V_BLK, EMBED_D), lambda i: (i, 0)),
            pl.BlockSpec((BATCH, EMBED_D), lambda i: (0, 0)),
        ],
        out_specs=pl.BlockSpec((BATCH, V_BLK), lambda i: (0, i)),
        out_shape=jax.ShapeDtypeStruct((BATCH, VOCAB), jnp.float32),
    )(embeddings, embed)


def kernel(target, embeddings):
    idx = target.astype(jnp.int32)
    embed = _build_sc_gather()(embeddings, idx)
    return _projection(embeddings, embed)


# speedup vs baseline: 2.9978x; 2.9978x over previous
import functools

import jax
import jax.numpy as jnp
from jax import lax
from jax.experimental import pallas as pl
from jax.experimental.pallas import tpu as pltpu
from jax.experimental.pallas import tpu_sc as plsc

VOCAB = 100000
EMBED_D = 16
BATCH = 1024

V_BLK = 2048


@functools.cache
def _build_sc_gather():
    info = plsc.get_sparse_core_info()
    nc, ns = info.num_cores, info.num_subcores
    nw = nc * ns
    b_per_w = BATCH // nw

    mesh = plsc.VectorSubcoreMesh(core_axis_name="c", subcore_axis_name="s")

    @functools.partial(
        pl.kernel,
        mesh=mesh,
        out_type=jax.ShapeDtypeStruct((BATCH, EMBED_D), jnp.float32),
        scratch_types=[
            pltpu.VMEM((b_per_w,), jnp.int32),
            pltpu.VMEM((b_per_w, EMBED_D), jnp.float32),
            pltpu.SemaphoreType.DMA,
        ],
        compiler_params=pltpu.CompilerParams(use_tc_tiling_on_sc=False),
    )
    def gather(table_hbm, idx_hbm, out_hbm, idx_v, rows_v, sem):
        wid = lax.axis_index("s") * nc + lax.axis_index("c")
        base = wid * b_per_w
        pltpu.sync_copy(idx_hbm.at[pl.ds(base, b_per_w)], idx_v)
        pltpu.async_copy(table_hbm.at[idx_v], rows_v, sem).wait()
        pltpu.sync_copy(rows_v, out_hbm.at[pl.ds(base, b_per_w)])

    return gather


def _matmul_body(embT_ref, vecT_ref, out_ref):
    lhs = embT_ref[...].astype(jnp.bfloat16)
    rhs = vecT_ref[...].astype(jnp.bfloat16)
    out_ref[...] = lax.dot_general(
        lhs,
        rhs,
        (((0,), (0,)), ((), ())),
        preferred_element_type=jnp.float32,
    )


def _projection_t(embT, vecT):
    grid = (pl.cdiv(VOCAB, V_BLK),)
    return pl.pallas_call(
        _matmul_body,
        grid=grid,
        in_specs=[
            pl.BlockSpec((EMBED_D, V_BLK), lambda i: (0, i)),
            pl.BlockSpec((EMBED_D, BATCH), lambda i: (0, 0)),
        ],
        out_specs=pl.BlockSpec((V_BLK, BATCH), lambda i: (i, 0)),
        out_shape=jax.ShapeDtypeStruct((VOCAB, BATCH), jnp.float32),
    )(embT, vecT)


def kernel(target, embeddings):
    idx = target.astype(jnp.int32)
    embed = _build_sc_gather()(embeddings, idx)
    logits_t = _projection_t(embeddings.T, embed.T)
    return logits_t.T


# speedup vs baseline: 3.7011x; 1.2346x over previous
import functools

import jax
import jax.numpy as jnp
from jax import lax
from jax.experimental import pallas as pl
from jax.experimental.pallas import tpu as pltpu
from jax.experimental.pallas import tpu_sc as plsc

VOCAB = 100000
EMBED_D = 16
BATCH = 1024

V_BLK = 2048


@functools.cache
def _build_sc_gather():
    info = plsc.get_sparse_core_info()
    nc, ns, lanes = info.num_cores, info.num_subcores, info.num_lanes
    nw = nc * ns
    b_per_w = BATCH // nw
    n_grp = b_per_w // lanes

    mesh = plsc.VectorSubcoreMesh(core_axis_name="c", subcore_axis_name="s")

    n_idx = b_per_w * EMBED_D

    @functools.partial(
        pl.kernel,
        mesh=mesh,
        out_type=jax.ShapeDtypeStruct((BATCH * EMBED_D,), jnp.float32),
        scratch_types=[
            pltpu.VMEM((b_per_w,), jnp.int32),
            pltpu.VMEM((n_idx,), jnp.int32),
            pltpu.VMEM((n_idx,), jnp.float32),
            pltpu.SemaphoreType.DMA,
        ],
        compiler_params=pltpu.CompilerParams(use_tc_tiling_on_sc=False),
    )
    def gather(flat_hbm, idx_hbm, out_hbm, idx_v, rows_v, vals_v, sem):
        wid = lax.axis_index("s") * nc + lax.axis_index("c")
        base = wid * b_per_w
        pltpu.sync_copy(idx_hbm.at[pl.ds(base, b_per_w)], idx_v)
        d_iota = lax.iota(jnp.int32, lanes) * jnp.int32(VOCAB)
        for g in range(n_grp):
            tvec = idx_v[pl.ds(g * lanes, lanes)]
            for j in range(lanes):
                k = g * lanes + j
                rows_v[pl.ds(k * EMBED_D, EMBED_D)] = d_iota + tvec[j]
        copies = []
        for s in range(n_idx // 128):
            copies.append(
                pltpu.async_copy(
                    flat_hbm.at[rows_v.at[pl.ds(s * 128, 128)]],
                    vals_v.at[pl.ds(s * 128, 128)],
                    sem,
                )
            )
        for cp in copies:
            cp.wait()
        pltpu.sync_copy(vals_v, out_hbm.at[pl.ds(base * EMBED_D, n_idx)])

    return gather


def _matmul_body(embT_ref, vecT_ref, out_ref):
    lhs = embT_ref[...].astype(jnp.bfloat16)
    rhs = vecT_ref[...].astype(jnp.bfloat16)
    out_ref[...] = lax.dot_general(
        lhs,
        rhs,
        (((0,), (0,)), ((), ())),
        preferred_element_type=jnp.float32,
    )


def _projection_t(embT, vecT):
    grid = (pl.cdiv(VOCAB, V_BLK),)
    return pl.pallas_call(
        _matmul_body,
        grid=grid,
        in_specs=[
            pl.BlockSpec((EMBED_D, V_BLK), lambda i: (0, i)),
            pl.BlockSpec((EMBED_D, BATCH), lambda i: (0, 0)),
        ],
        out_specs=pl.BlockSpec((V_BLK, BATCH), lambda i: (i, 0)),
        out_shape=jax.ShapeDtypeStruct((VOCAB, BATCH), jnp.float32),
    )(embT, vecT)


def kernel(target, embeddings):
    idx = target.astype(jnp.int32)
    embt = embeddings.T
    embed_flat = _build_sc_gather()(embt.reshape(-1), idx)
    embed_t = embed_flat.reshape(BATCH, EMBED_D).T
    logits_t = _projection_t(embt, embed_t)
    return logits_t.T
